# baseline (speedup 1.0000x reference)
import jax
import jax.numpy as jnp
from jax import lax
from jax.experimental import pallas as pl
from jax.experimental.pallas import tpu as pltpu
from jax.experimental.pallas import tpu_sc as plsc

_VOCAB = 1000000
_DIM = 64
_B = 4096
_L = 200
_NC = 2
_NS = 16
_NW = _NC * _NS
_BLK = _B // _NW


def _transpose_unit(rows, tblk):
    d_idx = [lax.iota(jnp.int32, 16) + 16 * dg for dg in range(4)]

    @pl.loop(0, _BLK, unroll=4)
    def _(j):
        jv = jnp.full((16,), j, jnp.int32)
        for dg in range(4):
            x = plsc.load_gather(rows, [jv, d_idx[dg]])
            plsc.store_scatter(tblk, [d_idx[dg], jv], x)


def _gather_kernel(ids_hbm, table_hbm, out_hbm,
                   idx0, idx1, rows0, rows1, t0, t1,
                   gsem0, gsem1, wsem0, wsem1):
    idx = (idx0, idx1)
    rows = (rows0, rows1)
    tblk = (t0, t1)
    gsem = (gsem0, gsem1)
    wsem = (wsem0, wsem1)

    wid = lax.axis_index("s") * _NC + lax.axis_index("c")
    b0 = wid * _BLK

    def load_idx(p, l):
        pltpu.sync_copy(ids_hbm.at[l, pl.ds(b0, _BLK)], idx[p])

    def start_gather(p):
        pltpu.async_copy(table_hbm.at[idx[p]], rows[p], gsem[p])

    def wait_gather(p):
        pltpu.make_async_copy(table_hbm.at[idx[p]], rows[p], gsem[p]).wait()

    def start_write(p, l):
        pltpu.async_copy(tblk[p], out_hbm.at[l, :, pl.ds(b0, _BLK)], wsem[p])

    def wait_write(p, l):
        pltpu.make_async_copy(
            tblk[p], out_hbm.at[l, :, pl.ds(b0, _BLK)], wsem[p]).wait()

    for p in (0, 1):
        load_idx(p, p)
        start_gather(p)
    for p in (0, 1):
        wait_gather(p)
        _transpose_unit(rows[p], tblk[p])
        load_idx(p, p + 2)
        start_gather(p)
        start_write(p, p)

    def step(k, carry):
        for p in (0, 1):
            l = 2 * k + p
            wait_gather(p)
            wait_write(p, l)
            _transpose_unit(rows[p], tblk[p])
            load_idx(p, l + 2)
            start_gather(p)
            start_write(p, l)
        return carry

    lax.fori_loop(1, (_L - 2) // 2, step, 0)

    for p in (0, 1):
        l = _L - 2 + p
        wait_gather(p)
        wait_write(p, l)
        _transpose_unit(rows[p], tblk[p])
        start_write(p, l)
    for p in (0, 1):
        wait_write(p, _L - 2 + p)


@jax.jit
def kernel(input_ids, table):
    table128 = jnp.pad(table, ((0, 0), (0, 128 - _DIM)))
    ids_t = input_ids.T
    mesh = plsc.VectorSubcoreMesh(
        core_axis_name="c", subcore_axis_name="s",
        num_cores=_NC, num_subcores=_NS,
    )
    out3 = pl.kernel(
        _gather_kernel,
        out_type=jax.ShapeDtypeStruct((_L, _DIM, _B), jnp.float32),
        mesh=mesh,
        scratch_types=(
            [pltpu.VMEM((_BLK,), jnp.int32) for _ in range(2)]
            + [pltpu.VMEM((_BLK, 128), jnp.float32) for _ in range(2)]
            + [pltpu.VMEM((_DIM, _BLK), jnp.float32) for _ in range(2)]
            + [pltpu.SemaphoreType.DMA for _ in range(4)]
        ),
        compiler_params=pltpu.CompilerParams(needs_layout_passes=False),
    )(ids_t, table128)
    return out3.transpose(2, 0, 1)
